# baseline (speedup 1.0000x reference)
import functools

import jax
import jax.numpy as jnp
from jax import lax
from jax.experimental import pallas as pl
from jax.experimental.pallas import tpu as pltpu
from jax.experimental.pallas import tpu_sc as plsc

_N = 4096
_E = 131072
_LAYERS = 8
_DIM_PE = 64

_NC = 2
_NS = 16
_G16 = 16
_EPT = _E // _NS
_CH = 128
_NCH = _EPT // _CH
_ROWS = 256
_WIN = _ROWS * _N
_NPASS = _N // (_ROWS * _NC)
_TILE_WIN = _WIN // _NS
_ZW = 8192


def _sc_build_a(src, dst):
    mesh = plsc.VectorSubcoreMesh(core_axis_name="c", subcore_axis_name="s")

    @functools.partial(
        pl.kernel,
        mesh=mesh,
        out_type=jax.ShapeDtypeStruct((_N * _N,), jnp.float32),
        scratch_types=[
            pltpu.VMEM((_EPT,), jnp.int32),
            pltpu.VMEM((_EPT,), jnp.int32),
            pltpu.VMEM((_EPT,), jnp.int32),
            pltpu.VMEM((_EPT,), jnp.float32),
            pltpu.VMEM((_EPT,), jnp.int32),
            pltpu.VMEM((_EPT,), jnp.float32),
            pltpu.VMEM((_CH,), jnp.int32),
            pltpu.VMEM((_CH,), jnp.float32),
            pltpu.VMEM((_ZW,), jnp.float32),
            pltpu.VMEM_SHARED((_WIN,), jnp.float32),
            pltpu.SemaphoreType.DMA,
        ],
    )
    def build(src_h, dst_h, out_h, src_v, dst_v, gidx_v, val_v, widx_v,
              wval_v, fidx_v, fval_v, zero_v, win_s, sem):
        c = lax.axis_index("c")
        s = lax.axis_index("s")
        base = s * _EPT

        with jax.named_scope("sc_stage"):
            pltpu.sync_copy(src_h.at[pl.ds(base, _EPT)], src_v)
            pltpu.sync_copy(dst_h.at[pl.ds(base, _EPT)], dst_v)

        def zinit(i, carry):
            zero_v[pl.ds(i * _G16, _G16)] = jnp.zeros((_G16,), jnp.float32)
            return carry
        lax.fori_loop(0, _ZW // _G16, zinit, 0)
        for l in range(_CH // _G16):
            fidx_v[pl.ds(l * _G16, _G16)] = jnp.zeros((_G16,), jnp.int32)
            fval_v[pl.ds(l * _G16, _G16)] = jnp.zeros((_G16,), jnp.float32)

        def edgec(j, carry):
            for l in range(_CH // _G16):
                off = j * _CH + l * _G16
                s16 = src_v[pl.ds(off, _G16)]
                d16 = dst_v[pl.ds(off, _G16)]
                nsl = s16 != d16
                val_v[pl.ds(off, _G16)] = jnp.where(nsl, 1.0, 0.0)
                gidx_v[pl.ds(off, _G16)] = s16 * _N + d16
            return carry
        lax.fori_loop(0, _NCH, edgec, 0)

        for p in range(_NPASS):
            lo = (p * _NC + c) * _WIN
            with jax.named_scope("sc_zero"):
                zcps = [
                    pltpu.async_copy(
                        zero_v, win_s.at[pl.ds(s * _TILE_WIN + z * _ZW, _ZW)],
                        sem)
                    for z in range(_TILE_WIN // _ZW)
                ]
                for cp in zcps:
                    cp.wait()
                plsc.subcore_barrier()

            with jax.named_scope("sc_winc"):
                def winc(j, carry):
                    for l in range(_CH // _G16):
                        off = j * _CH + l * _G16
                        g16 = gidx_v[pl.ds(off, _G16)]
                        rel = g16 - lo
                        inw = (rel >= 0) & (rel < _WIN)
                        widx_v[pl.ds(off, _G16)] = jnp.where(inw, rel, 0)
                        wval_v[pl.ds(off, _G16)] = jnp.where(
                            inw, val_v[pl.ds(off, _G16)], 0.0)
                    return carry
                lax.fori_loop(0, _NCH, winc, 0)

            with jax.named_scope("sc_scatter"):
                cp = pltpu.async_copy(wval_v, win_s.at[widx_v], sem, add=True)
                fl = pltpu.async_copy(fval_v, win_s.at[fidx_v], sem, add=True)
                cp.wait()
                fl.wait()
                plsc.subcore_barrier()
                pl.delay(2000)

            with jax.named_scope("sc_out"):
                pltpu.sync_copy(win_s.at[pl.ds(s * _TILE_WIN, _TILE_WIN)],
                                out_h.at[pl.ds(lo + s * _TILE_WIN, _TILE_WIN)])

    return build(src, dst)


def _normalize(a, bm=512):
    def body(a_ref, o_ref):
        blk = a_ref[...]
        deg = jnp.maximum(jnp.sum(blk, axis=1, keepdims=True), 1.0)
        o_ref[...] = blk * (1.0 / deg)

    return pl.pallas_call(
        body,
        grid=(_N // bm,),
        in_specs=[pl.BlockSpec((bm, _N), lambda i: (i, 0))],
        out_specs=pl.BlockSpec((bm, _N), lambda i: (i, 0)),
        out_shape=jax.ShapeDtypeStruct((_N, _N), jnp.float32),
    )(a)


def _mm(a, b, bm=1024, bn=1024, bk=1024):
    m, k = a.shape
    _, n = b.shape

    def body(a_ref, b_ref, o_ref, acc_ref):
        kk = pl.program_id(2)

        @pl.when(kk == 0)
        def _init():
            acc_ref[...] = jnp.zeros_like(acc_ref)

        acc_ref[...] += lax.dot_general(
            a_ref[...], b_ref[...], (((1,), (0,)), ((), ())),
            preferred_element_type=jnp.float32)

        @pl.when(kk == pl.num_programs(2) - 1)
        def _out():
            o_ref[...] = acc_ref[...]

    return pl.pallas_call(
        body,
        grid=(m // bm, n // bn, k // bk),
        in_specs=[pl.BlockSpec((bm, bk), lambda i, j, kk: (i, kk)),
                  pl.BlockSpec((bk, bn), lambda i, j, kk: (kk, j))],
        out_specs=pl.BlockSpec((bm, bn), lambda i, j, kk: (i, j)),
        out_shape=jax.ShapeDtypeStruct((m, n), jnp.float32),
        scratch_shapes=[pltpu.VMEM((bm, bn), jnp.float32)],
        compiler_params=pltpu.CompilerParams(
            dimension_semantics=("parallel", "parallel", "arbitrary")),
    )(a, b)


def _diag_stack(P, P2, P3, P4, bm=256):
    def body(pr, p4r, pc, p2c, p3c, p4c, o_ref):
        i = pl.program_id(0)
        r_iota = lax.broadcasted_iota(jnp.int32, (_N, bm), 0)
        c_iota = lax.broadcasted_iota(jnp.int32, (_N, bm), 1)
        dmask = r_iota == c_iota + i * bm

        def coldiag(col):
            return jnp.sum(jnp.where(dmask, col[...], 0.0), axis=0)

        eye = (lax.broadcasted_iota(jnp.int32, (bm, bm), 0) ==
               lax.broadcasted_iota(jnp.int32, (bm, bm), 1))

        def ddot(r, ccol):
            prod = lax.dot_general(
                r[...], ccol[...], (((1,), (0,)), ((), ())),
                preferred_element_type=jnp.float32)
            return jnp.sum(jnp.where(eye, prod, 0.0), axis=1)

        d1 = coldiag(pc)
        d2 = ddot(pr, pc)
        d3 = coldiag(p3c)
        d4 = coldiag(p4c)
        d5 = ddot(p4r, pc)
        d6 = ddot(p4r, p2c)
        d7 = ddot(p4r, p3c)
        d8 = ddot(p4r, p4c)
        o_ref[...] = jnp.stack([d1, d2, d3, d4, d5, d6, d7, d8], axis=1)

    def rowspec():
        return pl.BlockSpec((bm, _N), lambda i: (i, 0))

    def colspec():
        return pl.BlockSpec((_N, bm), lambda i: (0, i))

    return pl.pallas_call(
        body,
        grid=(_N // bm,),
        in_specs=[rowspec(), rowspec(), colspec(), colspec(), colspec(),
                  colspec()],
        out_specs=pl.BlockSpec((bm, _LAYERS), lambda i: (i, 0)),
        out_shape=jax.ShapeDtypeStruct((_N, _LAYERS), jnp.float32),
    )(P, P4, P, P2, P3, P4)


def _finish(rwse, w_t, bias, gamma, beta):
    def body(x_ref, w_ref, b_ref, g_ref, bt_ref, o_ref):
        x = x_ref[...]
        mean = jnp.mean(x, axis=0, keepdims=True)
        var = jnp.mean((x - mean) ** 2, axis=0, keepdims=True)
        xn = (x - mean) / jnp.sqrt(var + 1e-5) * g_ref[...] + bt_ref[...]
        o_ref[...] = lax.dot_general(
            xn, w_ref[...], (((1,), (0,)), ((), ())),
            preferred_element_type=jnp.float32) + b_ref[...]

    return pl.pallas_call(
        body,
        out_shape=jax.ShapeDtypeStruct((_N, _DIM_PE), jnp.float32),
    )(rwse, w_t, bias, gamma, beta)


def kernel(batch_vec, edge_index, EigVals, EigVecs, lin_w, lin_b, bn_gamma,
           bn_beta):
    src = edge_index[0]
    dst = edge_index[1]
    a_flat = _sc_build_a(src, dst)
    P = _normalize(a_flat.reshape(_N, _N))
    P2 = _mm(P, P)
    P3 = _mm(P2, P)
    P4 = _mm(P2, P2)
    rwse = _diag_stack(P, P2, P3, P4)
    return _finish(rwse, lin_w.T, lin_b.reshape(1, _DIM_PE),
                   bn_gamma.reshape(1, _LAYERS), bn_beta.reshape(1, _LAYERS))


# speedup vs baseline: 2.0781x; 2.0781x over previous
import functools

import jax
import jax.numpy as jnp
from jax import lax
from jax.experimental import pallas as pl
from jax.experimental.pallas import tpu as pltpu
from jax.experimental.pallas import tpu_sc as plsc

_N = 4096
_E = 131072
_LAYERS = 8
_DIM_PE = 64

_NC = 2
_NS = 16
_G16 = 16
_EPT = _E // _NS
_CH = 128
_NCH = _EPT // _CH
_ROWS = 256
_WIN = _ROWS * _N
_NPASS = _N // (_ROWS * _NC)
_TILE_WIN = _WIN // _NS
_ZW = 8192


def _sc_build_a(src, dst):
    mesh = plsc.VectorSubcoreMesh(core_axis_name="c", subcore_axis_name="s")

    @functools.partial(
        pl.kernel,
        mesh=mesh,
        out_type=jax.ShapeDtypeStruct((_N * _N,), jnp.float32),
        scratch_types=[
            pltpu.VMEM((_EPT,), jnp.int32),
            pltpu.VMEM((_EPT,), jnp.int32),
            pltpu.VMEM((_EPT,), jnp.int32),
            pltpu.VMEM((_EPT,), jnp.float32),
            pltpu.VMEM((_EPT,), jnp.int32),
            pltpu.VMEM((_EPT,), jnp.float32),
            pltpu.VMEM((_CH,), jnp.int32),
            pltpu.VMEM((_CH,), jnp.float32),
            pltpu.VMEM((_ZW,), jnp.float32),
            pltpu.VMEM_SHARED((_WIN + _EPT,), jnp.float32),
            pltpu.SemaphoreType.DMA,
        ],
    )
    def build(src_h, dst_h, out_h, src_v, dst_v, gidx_v, val_v, widx_v,
              wval_v, fidx_v, fval_v, zero_v, win_s, sem):
        c = lax.axis_index("c")
        s = lax.axis_index("s")
        base = s * _EPT

        with jax.named_scope("sc_stage"):
            pltpu.sync_copy(src_h.at[pl.ds(base, _EPT)], src_v)
            pltpu.sync_copy(dst_h.at[pl.ds(base, _EPT)], dst_v)

        def zinit(i, carry):
            zero_v[pl.ds(i * _G16, _G16)] = jnp.zeros((_G16,), jnp.float32)
            return carry
        lax.fori_loop(0, _ZW // _G16, zinit, 0)
        for l in range(_CH // _G16):
            fidx_v[pl.ds(l * _G16, _G16)] = jnp.zeros((_G16,), jnp.int32)
            fval_v[pl.ds(l * _G16, _G16)] = jnp.zeros((_G16,), jnp.float32)

        def edgec(j, carry):
            for l in range(_CH // _G16):
                off = j * _CH + l * _G16
                s16 = src_v[pl.ds(off, _G16)]
                d16 = dst_v[pl.ds(off, _G16)]
                nsl = s16 != d16
                val_v[pl.ds(off, _G16)] = jnp.where(nsl, 1.0, 0.0)
                gidx_v[pl.ds(off, _G16)] = s16 * _N + d16
            return carry
        lax.fori_loop(0, _NCH, edgec, 0)

        for p in range(_NPASS):
            lo = (p * _NC + c) * _WIN
            with jax.named_scope("sc_zero"):
                zcps = [
                    pltpu.async_copy(
                        zero_v, win_s.at[pl.ds(s * _TILE_WIN + z * _ZW, _ZW)],
                        sem)
                    for z in range(_TILE_WIN // _ZW)
                ]
                for cp in zcps:
                    cp.wait()
                plsc.subcore_barrier()

            with jax.named_scope("sc_winc"):
                def winc(j, carry):
                    for l in range(_CH // _G16):
                        off = j * _CH + l * _G16
                        g16 = gidx_v[pl.ds(off, _G16)]
                        rel = g16 - lo
                        inw = (rel >= 0) & (rel < _WIN)
                        widx_v[pl.ds(off, _G16)] = jnp.where(
                            inw, rel, _WIN + (g16 & (_EPT - 1)))
                        wval_v[pl.ds(off, _G16)] = jnp.where(
                            inw, val_v[pl.ds(off, _G16)], 0.0)
                    return carry
                lax.fori_loop(0, _NCH, winc, 0)

            with jax.named_scope("sc_scatter"):
                cp = pltpu.async_copy(wval_v, win_s.at[widx_v], sem, add=True)
                fl = pltpu.async_copy(fval_v, win_s.at[fidx_v], sem, add=True)
                cp.wait()
                fl.wait()
                plsc.subcore_barrier()
                pl.delay(2000)

            with jax.named_scope("sc_out"):
                pltpu.sync_copy(win_s.at[pl.ds(s * _TILE_WIN, _TILE_WIN)],
                                out_h.at[pl.ds(lo + s * _TILE_WIN, _TILE_WIN)])

    return build(src, dst)


def _normalize(a, bm=512):
    def body(a_ref, o_ref):
        blk = a_ref[...]
        deg = jnp.maximum(jnp.sum(blk, axis=1, keepdims=True), 1.0)
        o_ref[...] = blk * (1.0 / deg)

    return pl.pallas_call(
        body,
        grid=(_N // bm,),
        in_specs=[pl.BlockSpec((bm, _N), lambda i: (i, 0))],
        out_specs=pl.BlockSpec((bm, _N), lambda i: (i, 0)),
        out_shape=jax.ShapeDtypeStruct((_N, _N), jnp.float32),
    )(a)


def _mm(a, b, bm=1024, bn=1024, bk=1024):
    m, k = a.shape
    _, n = b.shape

    def body(a_ref, b_ref, o_ref, acc_ref):
        kk = pl.program_id(2)

        @pl.when(kk == 0)
        def _init():
            acc_ref[...] = jnp.zeros_like(acc_ref)

        acc_ref[...] += lax.dot_general(
            a_ref[...], b_ref[...], (((1,), (0,)), ((), ())),
            preferred_element_type=jnp.float32)

        @pl.when(kk == pl.num_programs(2) - 1)
        def _out():
            o_ref[...] = acc_ref[...]

    return pl.pallas_call(
        body,
        grid=(m // bm, n // bn, k // bk),
        in_specs=[pl.BlockSpec((bm, bk), lambda i, j, kk: (i, kk)),
                  pl.BlockSpec((bk, bn), lambda i, j, kk: (kk, j))],
        out_specs=pl.BlockSpec((bm, bn), lambda i, j, kk: (i, j)),
        out_shape=jax.ShapeDtypeStruct((m, n), jnp.float32),
        scratch_shapes=[pltpu.VMEM((bm, bn), jnp.float32)],
        compiler_params=pltpu.CompilerParams(
            dimension_semantics=("parallel", "parallel", "arbitrary")),
    )(a, b)


def _diag_stack(P, P2, P3, P4, bm=256):
    def body(pr, p4r, pc, p2c, p3c, p4c, o_ref):
        i = pl.program_id(0)
        r_iota = lax.broadcasted_iota(jnp.int32, (_N, bm), 0)
        c_iota = lax.broadcasted_iota(jnp.int32, (_N, bm), 1)
        dmask = r_iota == c_iota + i * bm

        def coldiag(col):
            return jnp.sum(jnp.where(dmask, col[...], 0.0), axis=0)

        eye = (lax.broadcasted_iota(jnp.int32, (bm, bm), 0) ==
               lax.broadcasted_iota(jnp.int32, (bm, bm), 1))

        def ddot(r, ccol):
            prod = lax.dot_general(
                r[...], ccol[...], (((1,), (0,)), ((), ())),
                preferred_element_type=jnp.float32)
            return jnp.sum(jnp.where(eye, prod, 0.0), axis=1)

        d1 = coldiag(pc)
        d2 = ddot(pr, pc)
        d3 = coldiag(p3c)
        d4 = coldiag(p4c)
        d5 = ddot(p4r, pc)
        d6 = ddot(p4r, p2c)
        d7 = ddot(p4r, p3c)
        d8 = ddot(p4r, p4c)
        o_ref[...] = jnp.stack([d1, d2, d3, d4, d5, d6, d7, d8], axis=1)

    def rowspec():
        return pl.BlockSpec((bm, _N), lambda i: (i, 0))

    def colspec():
        return pl.BlockSpec((_N, bm), lambda i: (0, i))

    return pl.pallas_call(
        body,
        grid=(_N // bm,),
        in_specs=[rowspec(), rowspec(), colspec(), colspec(), colspec(),
                  colspec()],
        out_specs=pl.BlockSpec((bm, _LAYERS), lambda i: (i, 0)),
        out_shape=jax.ShapeDtypeStruct((_N, _LAYERS), jnp.float32),
    )(P, P4, P, P2, P3, P4)


def _finish(rwse, w_t, bias, gamma, beta):
    def body(x_ref, w_ref, b_ref, g_ref, bt_ref, o_ref):
        x = x_ref[...]
        mean = jnp.mean(x, axis=0, keepdims=True)
        var = jnp.mean((x - mean) ** 2, axis=0, keepdims=True)
        xn = (x - mean) / jnp.sqrt(var + 1e-5) * g_ref[...] + bt_ref[...]
        o_ref[...] = lax.dot_general(
            xn, w_ref[...], (((1,), (0,)), ((), ())),
            preferred_element_type=jnp.float32) + b_ref[...]

    return pl.pallas_call(
        body,
        out_shape=jax.ShapeDtypeStruct((_N, _DIM_PE), jnp.float32),
    )(rwse, w_t, bias, gamma, beta)


def kernel(batch_vec, edge_index, EigVals, EigVecs, lin_w, lin_b, bn_gamma,
           bn_beta):
    src = edge_index[0]
    dst = edge_index[1]
    a_flat = _sc_build_a(src, dst)
    P = _normalize(a_flat.reshape(_N, _N))
    P2 = _mm(P, P)
    P3 = _mm(P2, P)
    P4 = _mm(P2, P2)
    rwse = _diag_stack(P, P2, P3, P4)
    return _finish(rwse, lin_w.T, lin_b.reshape(1, _DIM_PE),
                   bn_gamma.reshape(1, _LAYERS), bn_beta.reshape(1, _LAYERS))


# speedup vs baseline: 2.4026x; 1.1562x over previous
import functools

import jax
import jax.numpy as jnp
from jax import lax
from jax.experimental import pallas as pl
from jax.experimental.pallas import tpu as pltpu
from jax.experimental.pallas import tpu_sc as plsc

_N = 4096
_E = 131072
_LAYERS = 8
_DIM_PE = 64

_NC = 2
_NS = 16
_G16 = 16
_EPT = _E // _NS
_CH = 128
_NCH = _EPT // _CH
_ROWS = 256
_WIN = _ROWS * _N
_NPASS = _N // (_ROWS * _NC)
_TILE_WIN = _WIN // _NS
_ZW = 8192


def _sc_build_a(src, dst):
    mesh = plsc.VectorSubcoreMesh(core_axis_name="c", subcore_axis_name="s")

    @functools.partial(
        pl.kernel,
        mesh=mesh,
        out_type=jax.ShapeDtypeStruct((_N * _N,), jnp.float32),
        scratch_types=[
            pltpu.VMEM((_EPT,), jnp.int32),
            pltpu.VMEM((_EPT,), jnp.int32),
            pltpu.VMEM((_EPT,), jnp.int32),
            pltpu.VMEM((_EPT,), jnp.float32),
            pltpu.VMEM((_EPT,), jnp.int32),
            pltpu.VMEM((_EPT,), jnp.float32),
            pltpu.VMEM((_CH,), jnp.int32),
            pltpu.VMEM((_CH,), jnp.float32),
            pltpu.VMEM((_ZW,), jnp.float32),
            pltpu.VMEM_SHARED((_WIN + _EPT,), jnp.float32),
            pltpu.SemaphoreType.DMA,
        ],
    )
    def build(src_h, dst_h, out_h, src_v, dst_v, gidx_v, val_v, widx_v,
              wval_v, fidx_v, fval_v, zero_v, win_s, sem):
        c = lax.axis_index("c")
        s = lax.axis_index("s")
        base = s * _EPT

        with jax.named_scope("sc_stage"):
            pltpu.sync_copy(src_h.at[pl.ds(base, _EPT)], src_v)
            pltpu.sync_copy(dst_h.at[pl.ds(base, _EPT)], dst_v)

        def zinit(i, carry):
            zero_v[pl.ds(i * _G16, _G16)] = jnp.zeros((_G16,), jnp.float32)
            return carry
        lax.fori_loop(0, _ZW // _G16, zinit, 0)
        for l in range(_CH // _G16):
            fidx_v[pl.ds(l * _G16, _G16)] = jnp.zeros((_G16,), jnp.int32)
            fval_v[pl.ds(l * _G16, _G16)] = jnp.zeros((_G16,), jnp.float32)

        def edgec(j, carry):
            for l in range(_CH // _G16):
                off = j * _CH + l * _G16
                s16 = src_v[pl.ds(off, _G16)]
                d16 = dst_v[pl.ds(off, _G16)]
                nsl = s16 != d16
                val_v[pl.ds(off, _G16)] = jnp.where(nsl, 1.0, 0.0)
                gidx_v[pl.ds(off, _G16)] = s16 * _N + d16
            return carry
        lax.fori_loop(0, _NCH, edgec, 0)

        for p in range(_NPASS):
            lo = (p * _NC + c) * _WIN
            with jax.named_scope("sc_zero"):
                zcps = [
                    pltpu.async_copy(
                        zero_v, win_s.at[pl.ds(s * _TILE_WIN + z * _ZW, _ZW)],
                        sem)
                    for z in range(_TILE_WIN // _ZW)
                ]
                for cp in zcps:
                    cp.wait()
                plsc.subcore_barrier()

            with jax.named_scope("sc_winc"):
                def winc(j, carry):
                    for l in range(_CH // _G16):
                        off = j * _CH + l * _G16
                        g16 = gidx_v[pl.ds(off, _G16)]
                        rel = g16 - lo
                        inw = (rel >= 0) & (rel < _WIN)
                        widx_v[pl.ds(off, _G16)] = jnp.where(
                            inw, rel, _WIN + (g16 & (_EPT - 1)))
                        wval_v[pl.ds(off, _G16)] = jnp.where(
                            inw, val_v[pl.ds(off, _G16)], 0.0)
                    return carry
                lax.fori_loop(0, _NCH, winc, 0)

            with jax.named_scope("sc_scatter"):
                cp = pltpu.async_copy(wval_v, win_s.at[widx_v], sem, add=True)
                fl = pltpu.async_copy(fval_v, win_s.at[fidx_v], sem, add=True)
                cp.wait()
                fl.wait()
                plsc.subcore_barrier()
                pl.delay(2000)

            with jax.named_scope("sc_out"):
                pltpu.sync_copy(win_s.at[pl.ds(s * _TILE_WIN, _TILE_WIN)],
                                out_h.at[pl.ds(lo + s * _TILE_WIN, _TILE_WIN)])

    return build(src, dst)


def _normalize(a, bm=512):
    def body(a_ref, o_ref):
        blk = a_ref[...]
        deg = jnp.maximum(jnp.sum(blk, axis=1, keepdims=True), 1.0)
        o_ref[...] = blk * (1.0 / deg)

    return pl.pallas_call(
        body,
        grid=(_N // bm,),
        in_specs=[pl.BlockSpec((bm, _N), lambda i: (i, 0))],
        out_specs=pl.BlockSpec((bm, _N), lambda i: (i, 0)),
        out_shape=jax.ShapeDtypeStruct((_N, _N), jnp.float32),
    )(a)


def _mm(a, b, bm=2048, bn=2048, bk=512):
    m, k = a.shape
    _, n = b.shape

    def body(a_ref, b_ref, o_ref):
        kk = pl.program_id(2)

        @pl.when(kk == 0)
        def _init():
            o_ref[...] = jnp.zeros_like(o_ref)

        o_ref[...] += lax.dot_general(
            a_ref[...], b_ref[...], (((1,), (0,)), ((), ())),
            preferred_element_type=jnp.float32)

    return pl.pallas_call(
        body,
        grid=(m // bm, n // bn, k // bk),
        in_specs=[pl.BlockSpec((bm, bk), lambda i, j, kk: (i, kk)),
                  pl.BlockSpec((bk, bn), lambda i, j, kk: (kk, j))],
        out_specs=pl.BlockSpec((bm, bn), lambda i, j, kk: (i, j)),
        out_shape=jax.ShapeDtypeStruct((m, n), jnp.float32),
        compiler_params=pltpu.CompilerParams(
            dimension_semantics=("parallel", "parallel", "arbitrary")),
    )(a, b)


def _diag_stack(P, P2, P3, P4, bm=256):
    def body(pr, p4r, pc, p2c, p3c, p4c, o_ref):
        i = pl.program_id(0)
        r_iota = lax.broadcasted_iota(jnp.int32, (_N, bm), 0)
        c_iota = lax.broadcasted_iota(jnp.int32, (_N, bm), 1)
        dmask = r_iota == c_iota + i * bm

        def coldiag(col):
            return jnp.sum(jnp.where(dmask, col[...], 0.0), axis=0)

        eye = (lax.broadcasted_iota(jnp.int32, (bm, bm), 0) ==
               lax.broadcasted_iota(jnp.int32, (bm, bm), 1))

        def ddot(r, ccol):
            prod = lax.dot_general(
                r[...], ccol[...], (((1,), (0,)), ((), ())),
                preferred_element_type=jnp.float32)
            return jnp.sum(jnp.where(eye, prod, 0.0), axis=1)

        d1 = coldiag(pc)
        d2 = ddot(pr, pc)
        d3 = coldiag(p3c)
        d4 = coldiag(p4c)
        d5 = ddot(p4r, pc)
        d6 = ddot(p4r, p2c)
        d7 = ddot(p4r, p3c)
        d8 = ddot(p4r, p4c)
        o_ref[...] = jnp.stack([d1, d2, d3, d4, d5, d6, d7, d8], axis=1)

    def rowspec():
        return pl.BlockSpec((bm, _N), lambda i: (i, 0))

    def colspec():
        return pl.BlockSpec((_N, bm), lambda i: (0, i))

    return pl.pallas_call(
        body,
        grid=(_N // bm,),
        in_specs=[rowspec(), rowspec(), colspec(), colspec(), colspec(),
                  colspec()],
        out_specs=pl.BlockSpec((bm, _LAYERS), lambda i: (i, 0)),
        out_shape=jax.ShapeDtypeStruct((_N, _LAYERS), jnp.float32),
    )(P, P4, P, P2, P3, P4)


def _finish(rwse, w_t, bias, gamma, beta):
    def body(x_ref, w_ref, b_ref, g_ref, bt_ref, o_ref):
        x = x_ref[...]
        mean = jnp.mean(x, axis=0, keepdims=True)
        var = jnp.mean((x - mean) ** 2, axis=0, keepdims=True)
        xn = (x - mean) / jnp.sqrt(var + 1e-5) * g_ref[...] + bt_ref[...]
        o_ref[...] = lax.dot_general(
            xn, w_ref[...], (((1,), (0,)), ((), ())),
            preferred_element_type=jnp.float32) + b_ref[...]

    return pl.pallas_call(
        body,
        out_shape=jax.ShapeDtypeStruct((_N, _DIM_PE), jnp.float32),
    )(rwse, w_t, bias, gamma, beta)


def kernel(batch_vec, edge_index, EigVals, EigVecs, lin_w, lin_b, bn_gamma,
           bn_beta):
    src = edge_index[0]
    dst = edge_index[1]
    a_flat = _sc_build_a(src, dst)
    P = _normalize(a_flat.reshape(_N, _N))
    P2 = _mm(P, P)
    P3 = _mm(P2, P)
    P4 = _mm(P2, P2)
    rwse = _diag_stack(P, P2, P3, P4)
    return _finish(rwse, lin_w.T, lin_b.reshape(1, _DIM_PE),
                   bn_gamma.reshape(1, _LAYERS), bn_beta.reshape(1, _LAYERS))


# speedup vs baseline: 2.6327x; 1.0958x over previous
import functools

import jax
import jax.numpy as jnp
from jax import lax
from jax.experimental import pallas as pl
from jax.experimental.pallas import tpu as pltpu
from jax.experimental.pallas import tpu_sc as plsc

_N = 4096
_E = 131072
_LAYERS = 8
_DIM_PE = 64

_NC = 2
_NS = 16
_G16 = 16
_EPT = _E // _NS
_CH = 128
_NCH = _EPT // _CH
_ROWS = 256
_WIN = _ROWS * _N
_NPASS = _N // (_ROWS * _NC)
_TILE_WIN = _WIN // _NS
_ZW = 8192


def _sc_build_a(src, dst):
    mesh = plsc.VectorSubcoreMesh(core_axis_name="c", subcore_axis_name="s")

    @functools.partial(
        pl.kernel,
        mesh=mesh,
        out_type=jax.ShapeDtypeStruct((_N * _N,), jnp.float32),
        scratch_types=[
            pltpu.VMEM((_EPT,), jnp.int32),
            pltpu.VMEM((_EPT,), jnp.int32),
            pltpu.VMEM((_EPT,), jnp.int32),
            pltpu.VMEM((_EPT,), jnp.float32),
            pltpu.VMEM((_EPT,), jnp.int32),
            pltpu.VMEM((_EPT,), jnp.float32),
            pltpu.VMEM((_CH,), jnp.int32),
            pltpu.VMEM((_CH,), jnp.float32),
            pltpu.VMEM((_ZW,), jnp.float32),
            pltpu.VMEM_SHARED((_WIN + _EPT,), jnp.float32),
            pltpu.SemaphoreType.DMA,
        ],
    )
    def build(src_h, dst_h, out_h, src_v, dst_v, gidx_v, val_v, widx_v,
              wval_v, fidx_v, fval_v, zero_v, win_s, sem):
        c = lax.axis_index("c")
        s = lax.axis_index("s")
        base = s * _EPT

        with jax.named_scope("sc_stage"):
            pltpu.sync_copy(src_h.at[pl.ds(base, _EPT)], src_v)
            pltpu.sync_copy(dst_h.at[pl.ds(base, _EPT)], dst_v)

        def zinit(i, carry):
            zero_v[pl.ds(i * _G16, _G16)] = jnp.zeros((_G16,), jnp.float32)
            return carry
        lax.fori_loop(0, _ZW // _G16, zinit, 0)
        for l in range(_CH // _G16):
            fidx_v[pl.ds(l * _G16, _G16)] = jnp.zeros((_G16,), jnp.int32)
            fval_v[pl.ds(l * _G16, _G16)] = jnp.zeros((_G16,), jnp.float32)

        def edgec(j, carry):
            for l in range(_CH // _G16):
                off = j * _CH + l * _G16
                s16 = src_v[pl.ds(off, _G16)]
                d16 = dst_v[pl.ds(off, _G16)]
                nsl = s16 != d16
                val_v[pl.ds(off, _G16)] = jnp.where(nsl, 1.0, 0.0)
                gidx_v[pl.ds(off, _G16)] = s16 * _N + d16
            return carry
        lax.fori_loop(0, _NCH, edgec, 0)

        for p in range(_NPASS):
            lo = (p * _NC + c) * _WIN
            with jax.named_scope("sc_zero"):
                zcps = [
                    pltpu.async_copy(
                        zero_v, win_s.at[pl.ds(s * _TILE_WIN + z * _ZW, _ZW)],
                        sem)
                    for z in range(_TILE_WIN // _ZW)
                ]
                for cp in zcps:
                    cp.wait()
                plsc.subcore_barrier()

            with jax.named_scope("sc_winc"):
                def winc(j, carry):
                    for l in range(_CH // _G16):
                        off = j * _CH + l * _G16
                        g16 = gidx_v[pl.ds(off, _G16)]
                        rel = g16 - lo
                        inw = (rel >= 0) & (rel < _WIN)
                        widx_v[pl.ds(off, _G16)] = jnp.where(
                            inw, rel, _WIN + (g16 & (_EPT - 1)))
                        wval_v[pl.ds(off, _G16)] = jnp.where(
                            inw, val_v[pl.ds(off, _G16)], 0.0)
                    return carry
                lax.fori_loop(0, _NCH, winc, 0)

            with jax.named_scope("sc_scatter"):
                cp = pltpu.async_copy(wval_v, win_s.at[widx_v], sem, add=True)
                fl = pltpu.async_copy(fval_v, win_s.at[fidx_v], sem, add=True)
                cp.wait()
                fl.wait()
                plsc.subcore_barrier()
                pl.delay(2000)

            with jax.named_scope("sc_out"):
                pltpu.sync_copy(win_s.at[pl.ds(s * _TILE_WIN, _TILE_WIN)],
                                out_h.at[pl.ds(lo + s * _TILE_WIN, _TILE_WIN)])

    return build(src, dst)


def _normalize(a, bm=512):
    def body(a_ref, o_ref):
        blk = a_ref[...]
        deg = jnp.maximum(jnp.sum(blk, axis=1, keepdims=True), 1.0)
        o_ref[...] = (blk * (1.0 / deg)).astype(jnp.bfloat16)

    return pl.pallas_call(
        body,
        grid=(_N // bm,),
        in_specs=[pl.BlockSpec((bm, _N), lambda i: (i, 0))],
        out_specs=pl.BlockSpec((bm, _N), lambda i: (i, 0)),
        out_shape=jax.ShapeDtypeStruct((_N, _N), jnp.bfloat16),
    )(a)


def _mm(a, b, bm=2048, bn=2048, bk=512):
    m, k = a.shape
    _, n = b.shape

    def body(a_ref, b_ref, o_ref, acc_ref):
        kk = pl.program_id(2)

        @pl.when(kk == 0)
        def _init():
            acc_ref[...] = jnp.zeros_like(acc_ref)

        acc_ref[...] += lax.dot_general(
            a_ref[...], b_ref[...], (((1,), (0,)), ((), ())),
            preferred_element_type=jnp.float32)

        @pl.when(kk == pl.num_programs(2) - 1)
        def _out():
            o_ref[...] = acc_ref[...].astype(jnp.bfloat16)

    return pl.pallas_call(
        body,
        grid=(m // bm, n // bn, k // bk),
        in_specs=[pl.BlockSpec((bm, bk), lambda i, j, kk: (i, kk)),
                  pl.BlockSpec((bk, bn), lambda i, j, kk: (kk, j))],
        out_specs=pl.BlockSpec((bm, bn), lambda i, j, kk: (i, j)),
        out_shape=jax.ShapeDtypeStruct((m, n), jnp.bfloat16),
        scratch_shapes=[pltpu.VMEM((bm, bn), jnp.float32)],
        compiler_params=pltpu.CompilerParams(
            dimension_semantics=("parallel", "parallel", "arbitrary")),
    )(a, b)


def _diag_stack(P, P2, P3, P4, bm=256):
    def body(pr, p4r, pc, p2c, p3c, p4c, o_ref):
        i = pl.program_id(0)
        r_iota = lax.broadcasted_iota(jnp.int32, (_N, bm), 0)
        c_iota = lax.broadcasted_iota(jnp.int32, (_N, bm), 1)
        dmask = r_iota == c_iota + i * bm

        def coldiag(col):
            return jnp.sum(
                jnp.where(dmask, col[...].astype(jnp.float32), 0.0), axis=0)

        eye = (lax.broadcasted_iota(jnp.int32, (bm, bm), 0) ==
               lax.broadcasted_iota(jnp.int32, (bm, bm), 1))

        def ddot(r, ccol):
            prod = lax.dot_general(
                r[...], ccol[...], (((1,), (0,)), ((), ())),
                preferred_element_type=jnp.float32)
            return jnp.sum(jnp.where(eye, prod, 0.0), axis=1)

        d1 = coldiag(pc)
        d2 = ddot(pr, pc)
        d3 = coldiag(p3c)
        d4 = coldiag(p4c)
        d5 = ddot(p4r, pc)
        d6 = ddot(p4r, p2c)
        d7 = ddot(p4r, p3c)
        d8 = ddot(p4r, p4c)
        o_ref[...] = jnp.stack([d1, d2, d3, d4, d5, d6, d7, d8], axis=1)

    def rowspec():
        return pl.BlockSpec((bm, _N), lambda i: (i, 0))

    def colspec():
        return pl.BlockSpec((_N, bm), lambda i: (0, i))

    return pl.pallas_call(
        body,
        grid=(_N // bm,),
        in_specs=[rowspec(), rowspec(), colspec(), colspec(), colspec(),
                  colspec()],
        out_specs=pl.BlockSpec((bm, _LAYERS), lambda i: (i, 0)),
        out_shape=jax.ShapeDtypeStruct((_N, _LAYERS), jnp.float32),
    )(P, P4, P, P2, P3, P4)


def _finish(rwse, w_t, bias, gamma, beta):
    def body(x_ref, w_ref, b_ref, g_ref, bt_ref, o_ref):
        x = x_ref[...]
        mean = jnp.mean(x, axis=0, keepdims=True)
        var = jnp.mean((x - mean) ** 2, axis=0, keepdims=True)
        xn = (x - mean) / jnp.sqrt(var + 1e-5) * g_ref[...] + bt_ref[...]
        o_ref[...] = lax.dot_general(
            xn, w_ref[...], (((1,), (0,)), ((), ())),
            preferred_element_type=jnp.float32) + b_ref[...]

    return pl.pallas_call(
        body,
        out_shape=jax.ShapeDtypeStruct((_N, _DIM_PE), jnp.float32),
    )(rwse, w_t, bias, gamma, beta)


def kernel(batch_vec, edge_index, EigVals, EigVecs, lin_w, lin_b, bn_gamma,
           bn_beta):
    src = edge_index[0]
    dst = edge_index[1]
    a_flat = _sc_build_a(src, dst)
    P = _normalize(a_flat.reshape(_N, _N))
    P2 = _mm(P, P)
    P3 = _mm(P2, P)
    P4 = _mm(P2, P2)
    rwse = _diag_stack(P, P2, P3, P4)
    return _finish(rwse, lin_w.T, lin_b.reshape(1, _DIM_PE),
                   bn_gamma.reshape(1, _LAYERS), bn_beta.reshape(1, _LAYERS))


# speedup vs baseline: 2.6495x; 1.0064x over previous
import functools

import jax
import jax.numpy as jnp
from jax import lax
from jax.experimental import pallas as pl
from jax.experimental.pallas import tpu as pltpu
from jax.experimental.pallas import tpu_sc as plsc

_N = 4096
_E = 131072
_LAYERS = 8
_DIM_PE = 64

_NC = 2
_NS = 16
_G16 = 16
_EPT = _E // _NS
_CH = 128
_NCH = _EPT // _CH
_ROWS = 256
_WIN = _ROWS * _N
_NPASS = _N // (_ROWS * _NC)
_TILE_WIN = _WIN // _NS
_ZW = 8192


def _sc_build_a(src, dst):
    mesh = plsc.VectorSubcoreMesh(core_axis_name="c", subcore_axis_name="s")

    @functools.partial(
        pl.kernel,
        mesh=mesh,
        out_type=jax.ShapeDtypeStruct((_N * _N,), jnp.float32),
        scratch_types=[
            pltpu.VMEM((_EPT,), jnp.int32),
            pltpu.VMEM((_EPT,), jnp.int32),
            pltpu.VMEM((_EPT,), jnp.int32),
            pltpu.VMEM((_EPT,), jnp.float32),
            pltpu.VMEM((_EPT,), jnp.int32),
            pltpu.VMEM((_EPT,), jnp.float32),
            pltpu.VMEM((_CH,), jnp.int32),
            pltpu.VMEM((_CH,), jnp.float32),
            pltpu.VMEM((_ZW,), jnp.float32),
            pltpu.VMEM_SHARED((_WIN + _EPT,), jnp.float32),
            pltpu.SemaphoreType.DMA,
        ],
    )
    def build(src_h, dst_h, out_h, src_v, dst_v, gidx_v, val_v, widx_v,
              wval_v, fidx_v, fval_v, zero_v, win_s, sem):
        c = lax.axis_index("c")
        s = lax.axis_index("s")
        base = s * _EPT

        with jax.named_scope("sc_stage"):
            pltpu.sync_copy(src_h.at[pl.ds(base, _EPT)], src_v)
            pltpu.sync_copy(dst_h.at[pl.ds(base, _EPT)], dst_v)

        def zinit(i, carry):
            zero_v[pl.ds(i * _G16, _G16)] = jnp.zeros((_G16,), jnp.float32)
            return carry
        lax.fori_loop(0, _ZW // _G16, zinit, 0)
        for l in range(_CH // _G16):
            fidx_v[pl.ds(l * _G16, _G16)] = jnp.zeros((_G16,), jnp.int32)
            fval_v[pl.ds(l * _G16, _G16)] = jnp.zeros((_G16,), jnp.float32)

        def edgec(j, carry):
            for l in range(_CH // _G16):
                off = j * _CH + l * _G16
                s16 = src_v[pl.ds(off, _G16)]
                d16 = dst_v[pl.ds(off, _G16)]
                nsl = s16 != d16
                val_v[pl.ds(off, _G16)] = jnp.where(nsl, 1.0, 0.0)
                gidx_v[pl.ds(off, _G16)] = s16 * _N + d16
            return carry
        lax.fori_loop(0, _NCH, edgec, 0)

        for p in range(_NPASS):
            lo = (p * _NC + c) * _WIN
            with jax.named_scope("sc_zero"):
                zcps = [
                    pltpu.async_copy(
                        zero_v, win_s.at[pl.ds(s * _TILE_WIN + z * _ZW, _ZW)],
                        sem)
                    for z in range(_TILE_WIN // _ZW)
                ]
                for cp in zcps:
                    cp.wait()
                plsc.subcore_barrier()

            with jax.named_scope("sc_winc"):
                def winc(j, carry):
                    for l in range(_CH // _G16):
                        off = j * _CH + l * _G16
                        g16 = gidx_v[pl.ds(off, _G16)]
                        rel = g16 - lo
                        inw = (rel >= 0) & (rel < _WIN)
                        widx_v[pl.ds(off, _G16)] = jnp.where(
                            inw, rel, _WIN + (g16 & (_EPT - 1)))
                        wval_v[pl.ds(off, _G16)] = jnp.where(
                            inw, val_v[pl.ds(off, _G16)], 0.0)
                    return carry
                lax.fori_loop(0, _NCH, winc, 0)

            with jax.named_scope("sc_scatter"):
                cp = pltpu.async_copy(wval_v, win_s.at[widx_v], sem, add=True)
                fl = pltpu.async_copy(fval_v, win_s.at[fidx_v], sem, add=True)
                cp.wait()
                fl.wait()
                plsc.subcore_barrier()
                pl.delay(2000)

            with jax.named_scope("sc_out"):
                pltpu.sync_copy(win_s.at[pl.ds(s * _TILE_WIN, _TILE_WIN)],
                                out_h.at[pl.ds(lo + s * _TILE_WIN, _TILE_WIN)])

    return build(src, dst)


def _normalize(a, bm=512):
    def body(a_ref, o_ref):
        blk = a_ref[...]
        deg = jnp.maximum(jnp.sum(blk, axis=1, keepdims=True), 1.0)
        o_ref[...] = (blk * (1.0 / deg)).astype(jnp.bfloat16)

    return pl.pallas_call(
        body,
        grid=(_N // bm,),
        in_specs=[pl.BlockSpec((bm, _N), lambda i: (i, 0))],
        out_specs=pl.BlockSpec((bm, _N), lambda i: (i, 0)),
        out_shape=jax.ShapeDtypeStruct((_N, _N), jnp.bfloat16),
    )(a)


def _mm(a, b, bm=2048, bn=2048, bk=1024):
    m, k = a.shape
    _, n = b.shape

    def body(a_ref, b_ref, o_ref, acc_ref):
        kk = pl.program_id(2)

        @pl.when(kk == 0)
        def _init():
            acc_ref[...] = jnp.zeros_like(acc_ref)

        acc_ref[...] += lax.dot_general(
            a_ref[...], b_ref[...], (((1,), (0,)), ((), ())),
            preferred_element_type=jnp.float32)

        @pl.when(kk == pl.num_programs(2) - 1)
        def _out():
            o_ref[...] = acc_ref[...].astype(jnp.bfloat16)

    return pl.pallas_call(
        body,
        grid=(m // bm, n // bn, k // bk),
        in_specs=[pl.BlockSpec((bm, bk), lambda i, j, kk: (i, kk)),
                  pl.BlockSpec((bk, bn), lambda i, j, kk: (kk, j))],
        out_specs=pl.BlockSpec((bm, bn), lambda i, j, kk: (i, j)),
        out_shape=jax.ShapeDtypeStruct((m, n), jnp.bfloat16),
        scratch_shapes=[pltpu.VMEM((bm, bn), jnp.float32)],
        compiler_params=pltpu.CompilerParams(
            dimension_semantics=("parallel", "parallel", "arbitrary")),
    )(a, b)


def _diag_stack(P, P2, P3, P4, bm=256):
    def body(pr, p4r, pc, p2c, p3c, p4c, o_ref):
        i = pl.program_id(0)
        r_iota = lax.broadcasted_iota(jnp.int32, (_N, bm), 0)
        c_iota = lax.broadcasted_iota(jnp.int32, (_N, bm), 1)
        dmask = r_iota == c_iota + i * bm

        def coldiag(col):
            return jnp.sum(
                jnp.where(dmask, col[...].astype(jnp.float32), 0.0), axis=0)

        eye = (lax.broadcasted_iota(jnp.int32, (bm, bm), 0) ==
               lax.broadcasted_iota(jnp.int32, (bm, bm), 1))

        def ddot(r, ccol):
            prod = lax.dot_general(
                r[...], ccol[...], (((1,), (0,)), ((), ())),
                preferred_element_type=jnp.float32)
            return jnp.sum(jnp.where(eye, prod, 0.0), axis=1)

        d1 = coldiag(pc)
        d2 = ddot(pr, pc)
        d3 = coldiag(p3c)
        d4 = coldiag(p4c)
        d5 = ddot(p4r, pc)
        d6 = ddot(p4r, p2c)
        d7 = ddot(p4r, p3c)
        d8 = ddot(p4r, p4c)
        o_ref[...] = jnp.stack([d1, d2, d3, d4, d5, d6, d7, d8], axis=1)

    def rowspec():
        return pl.BlockSpec((bm, _N), lambda i: (i, 0))

    def colspec():
        return pl.BlockSpec((_N, bm), lambda i: (0, i))

    return pl.pallas_call(
        body,
        grid=(_N // bm,),
        in_specs=[rowspec(), rowspec(), colspec(), colspec(), colspec(),
                  colspec()],
        out_specs=pl.BlockSpec((bm, _LAYERS), lambda i: (i, 0)),
        out_shape=jax.ShapeDtypeStruct((_N, _LAYERS), jnp.float32),
    )(P, P4, P, P2, P3, P4)


def _finish(rwse, w_t, bias, gamma, beta):
    def body(x_ref, w_ref, b_ref, g_ref, bt_ref, o_ref):
        x = x_ref[...]
        mean = jnp.mean(x, axis=0, keepdims=True)
        var = jnp.mean((x - mean) ** 2, axis=0, keepdims=True)
        xn = (x - mean) / jnp.sqrt(var + 1e-5) * g_ref[...] + bt_ref[...]
        o_ref[...] = lax.dot_general(
            xn, w_ref[...], (((1,), (0,)), ((), ())),
            preferred_element_type=jnp.float32) + b_ref[...]

    return pl.pallas_call(
        body,
        out_shape=jax.ShapeDtypeStruct((_N, _DIM_PE), jnp.float32),
    )(rwse, w_t, bias, gamma, beta)


def kernel(batch_vec, edge_index, EigVals, EigVecs, lin_w, lin_b, bn_gamma,
           bn_beta):
    src = edge_index[0]
    dst = edge_index[1]
    a_flat = _sc_build_a(src, dst)
    P = _normalize(a_flat.reshape(_N, _N))
    P2 = _mm(P, P)
    P3 = _mm(P2, P)
    P4 = _mm(P2, P2)
    rwse = _diag_stack(P, P2, P3, P4)
    return _finish(rwse, lin_w.T, lin_b.reshape(1, _DIM_PE),
                   bn_gamma.reshape(1, _LAYERS), bn_beta.reshape(1, _LAYERS))
